# baseline (speedup 1.0000x reference)
import functools
import math

import jax
import jax.numpy as jnp
from jax import lax
from jax.experimental import pallas as pl
from jax.experimental.pallas import tpu as pltpu
from jax.experimental.pallas import tpu_sc as plsc

ROWS = 2048
SIZE = 32000
PADDING_IDX = 0
SMOOTHING = 0.1
CONFIDENCE = 1.0 - SMOOTHING
EPS = SMOOTHING / (SIZE - 1)
C_CONST = (SIZE - 1) * EPS * math.log(EPS) + CONFIDENCE * math.log(CONFIDENCE)

L = 16
NW = 32

R_SC = 768
R_TC = ROWS - R_SC
BPWD = R_SC // NW


RB = 128
CB = SIZE
NI = R_TC // RB


def _rowsum_body(t_ref, x_ref, o_ref, acc_ref):
    i = pl.program_id(0)

    @pl.when(i == 0)
    def _init():
        acc_ref[0] = 0.0

    t = t_ref[0]
    m = (t != PADDING_IDX).astype(jnp.float32)
    xb = x_ref[...]
    part = jnp.dot(m, xb, preferred_element_type=jnp.float32)
    acc_ref[0] += jnp.sum(part)

    @pl.when(i == NI - 1)
    def _fin():
        o_ref[0, 0] = acc_ref[0]


def _masked_rowsum(x, target):
    t3 = target[:R_TC].reshape(NI, 1, RB)
    out = pl.pallas_call(
        _rowsum_body,
        grid=(NI,),
        in_specs=[
            pl.BlockSpec((1, 1, RB), lambda i: (i, 0, 0)),
            pl.BlockSpec((RB, CB), lambda i: (i, 0)),
        ],
        out_specs=pl.BlockSpec((1, 1), lambda i: (0, 0),
                               memory_space=pltpu.SMEM),
        out_shape=jax.ShapeDtypeStruct((1, 1), jnp.float32),
        scratch_shapes=[pltpu.SMEM((1,), jnp.float32)],
    )(t3, x)
    return out[0, 0]


BPW = ROWS // NW
NCHUNK = BPW // L


def _sc_kernel(tgt_hbm, xflat_hbm, mexp_hbm, b_hbm, tok_hbm, a_hbm,
               tgt_v, mexp_v, idx_v, val_v, out_v, row0, row1,
               semg, sem0, sem1):
    wid = lax.axis_index("s") * 2 + lax.axis_index("c")
    vzero = jnp.zeros((L,), jnp.float32)

    base = wid * BPW
    pltpu.sync_copy(tgt_hbm.at[pl.ds(base, BPW)], tgt_v)
    lanes = lax.iota(jnp.int32, L)
    for c in range(NCHUNK):
        t16 = tgt_v[pl.ds(c * L, L)]
        rows = base + c * L + lanes
        idx_v[pl.ds(c * L, L)] = rows * SIZE + t16
    gcp = pltpu.async_copy(xflat_hbm.at[idx_v], val_v, semg)

    dbase = R_TC + wid * BPWD
    pltpu.sync_copy(mexp_hbm.at[pl.ds(wid * BPWD * L, BPWD * L)], mexp_v)

    def row_sum(buf):
        def it(k, accs):
            a0, a1, a2, a3 = accs
            o = k * 64
            return (a0 + buf[pl.ds(o, L)],
                    a1 + buf[pl.ds(o + 16, L)],
                    a2 + buf[pl.ds(o + 32, L)],
                    a3 + buf[pl.ds(o + 48, L)])
        a0, a1, a2, a3 = lax.fori_loop(0, SIZE // 64, it, (vzero,) * 4)
        return (a0 + a1) + (a2 + a3)

    def mask_for(r):
        return mexp_v[pl.ds(r * L, L)]

    bufs = (row0, row1)
    sems = (sem0, sem1)
    cps = [None, None]
    cps[0] = pltpu.async_copy(
        xflat_hbm.at[pl.ds(dbase * SIZE, SIZE)], row0, sem0)
    aacc = vzero
    for r in range(BPWD):
        b = r % 2
        if r + 1 < BPWD:
            nb = (r + 1) % 2
            cps[nb] = pltpu.async_copy(
                xflat_hbm.at[pl.ds((dbase + r + 1) * SIZE, SIZE)],
                bufs[nb], sems[nb])
        cps[b].wait()
        aacc = aacc + row_sum(bufs[b]) * mask_for(r)
    out_v[...] = aacc
    pltpu.sync_copy(out_v, a_hbm.at[wid])

    gcp.wait()
    bacc = vzero
    tacc = vzero
    for c in range(NCHUNK):
        t16 = tgt_v[pl.ds(c * L, L)]
        v16 = val_v[pl.ds(c * L, L)]
        m = t16 != PADDING_IDX
        bacc = bacc + jnp.where(m, v16, 0.0)
        tacc = tacc + jnp.where(m, 1.0, 0.0)
    out_v[...] = bacc
    pltpu.sync_copy(out_v, b_hbm.at[wid])
    out_v[...] = tacc
    pltpu.sync_copy(out_v, tok_hbm.at[wid])


@functools.cache
def _make_sc_call():
    return functools.partial(
        pl.kernel,
        mesh=plsc.VectorSubcoreMesh(core_axis_name="c", subcore_axis_name="s"),
        out_type=[
            jax.ShapeDtypeStruct((NW, L), jnp.float32),
            jax.ShapeDtypeStruct((NW, L), jnp.float32),
            jax.ShapeDtypeStruct((NW, L), jnp.float32),
        ],
        scratch_types=[
            pltpu.VMEM((BPW,), jnp.int32),
            pltpu.VMEM((BPWD * L,), jnp.float32),
            pltpu.VMEM((BPW,), jnp.int32),
            pltpu.VMEM((BPW,), jnp.float32),
            pltpu.VMEM((L,), jnp.float32),
            pltpu.VMEM((SIZE,), jnp.float32),
            pltpu.VMEM((SIZE,), jnp.float32),
            pltpu.SemaphoreType.DMA,
            pltpu.SemaphoreType.DMA,
            pltpu.SemaphoreType.DMA,
        ],
    )(_sc_kernel)


def kernel(x, target):
    target = target.astype(jnp.int32)
    mexp = jnp.broadcast_to(
        (target[R_TC:] != PADDING_IDX).astype(jnp.float32)[:, None],
        (R_SC, L)).reshape(-1)
    b_parts, tok_parts, a_parts = _make_sc_call()(target, x.reshape(-1), mexp)
    a_tc = _masked_rowsum(x, target)
    a_sum = a_tc + jnp.sum(a_parts)
    b_sum = jnp.sum(b_parts)
    tokens = jnp.sum(tok_parts)
    c32 = jnp.float32(C_CONST)
    return c32 - (jnp.float32(EPS) * a_sum
                  + jnp.float32(CONFIDENCE - EPS) * b_sum) / tokens


# speedup vs baseline: 2.2672x; 2.2672x over previous
import functools
import math

import jax
import jax.numpy as jnp
from jax import lax
from jax.experimental import pallas as pl
from jax.experimental.pallas import tpu as pltpu
from jax.experimental.pallas import tpu_sc as plsc

ROWS = 2048
SIZE = 32000
PADDING_IDX = 0
SMOOTHING = 0.1
CONFIDENCE = 1.0 - SMOOTHING
EPS = SMOOTHING / (SIZE - 1)
C_CONST = (SIZE - 1) * EPS * math.log(EPS) + CONFIDENCE * math.log(CONFIDENCE)

L = 16
NW = 32

R_SC = 768
R_TC = ROWS - R_SC
BPWD = R_SC // NW


RB = 128
CB = SIZE
NI = R_TC // RB


def _tc_body(t_ref, x_ref, oa_ref, ob_ref, ot_ref, acc_ref):
    i = pl.program_id(0)

    @pl.when(i == 0)
    def _init():
        acc_ref[0] = 0.0
        acc_ref[1] = 0.0
        acc_ref[2] = 0.0

    t = t_ref[0]
    mcol = t != PADDING_IDX
    mf = mcol.astype(jnp.float32)
    xb = x_ref[...]
    colid = lax.broadcasted_iota(jnp.int32, (RB, CB), 1)
    acc_ref[0] += jnp.sum(xb * mf)
    acc_ref[1] += jnp.sum(jnp.where((colid == t) & mcol, xb, 0.0))
    acc_ref[2] += jnp.sum(mf)

    @pl.when(i == NI - 1)
    def _fin():
        oa_ref[0, 0] = acc_ref[0]
        ob_ref[0, 0] = acc_ref[1]
        ot_ref[0, 0] = acc_ref[2]


def _tc_part(x, target):
    t3 = target[:R_TC].reshape(NI, RB, 1)
    scalar_spec = pl.BlockSpec((1, 1), lambda i: (0, 0),
                               memory_space=pltpu.SMEM)
    outs = pl.pallas_call(
        _tc_body,
        grid=(NI,),
        in_specs=[
            pl.BlockSpec((1, RB, 1), lambda i: (i, 0, 0)),
            pl.BlockSpec((RB, CB), lambda i: (i, 0)),
        ],
        out_specs=[scalar_spec, scalar_spec, scalar_spec],
        out_shape=[jax.ShapeDtypeStruct((1, 1), jnp.float32)] * 3,
        scratch_shapes=[pltpu.SMEM((3,), jnp.float32)],
    )(t3, x)
    return outs[0][0, 0], outs[1][0, 0], outs[2][0, 0]


def _sc_kernel(x_hbm, texp_hbm, mexp_hbm, a_hbm, b_hbm, tok_hbm,
               texp_v, mexp_v, out_v, row0, row1, sem0, sem1):
    wid = lax.axis_index("s") * 2 + lax.axis_index("c")
    vzero = jnp.zeros((L,), jnp.float32)
    lanes = lax.iota(jnp.int32, L)
    dbase = R_TC + wid * BPWD
    pltpu.sync_copy(texp_hbm.at[pl.ds(wid * BPWD * L, BPWD * L)], texp_v)
    pltpu.sync_copy(mexp_hbm.at[pl.ds(wid * BPWD * L, BPWD * L)], mexp_v)

    def row_scan(buf, t16):
        def it(k, accs):
            a0, a1, a2, a3, g = accs
            o = k * 64
            c0 = buf[pl.ds(o, L)]
            c1 = buf[pl.ds(o + 16, L)]
            c2 = buf[pl.ds(o + 32, L)]
            c3 = buf[pl.ds(o + 48, L)]
            col0 = lanes + o
            g = g + jnp.where(col0 == t16, c0, 0.0)
            g = g + jnp.where(col0 + 16 == t16, c1, 0.0)
            g = g + jnp.where(col0 + 32 == t16, c2, 0.0)
            g = g + jnp.where(col0 + 48 == t16, c3, 0.0)
            return (a0 + c0, a1 + c1, a2 + c2, a3 + c3, g)
        a0, a1, a2, a3, g = lax.fori_loop(0, SIZE // 64, it, (vzero,) * 5)
        return (a0 + a1) + (a2 + a3), g

    bufs = (row0, row1)
    sems = (sem0, sem1)
    cps = [None, None]
    cps[0] = pltpu.async_copy(x_hbm.at[dbase], row0, sem0)
    aacc = vzero
    bacc = vzero
    tacc = vzero
    for r in range(BPWD):
        b = r % 2
        if r + 1 < BPWD:
            nb = (r + 1) % 2
            cps[nb] = pltpu.async_copy(
                x_hbm.at[dbase + r + 1], bufs[nb], sems[nb])
        cps[b].wait()
        t16 = texp_v[pl.ds(r * L, L)]
        mrow = mexp_v[pl.ds(r * L, L)]
        rowvec, g16 = row_scan(bufs[b], t16)
        aacc = aacc + rowvec * mrow
        bacc = bacc + g16 * mrow
        tacc = tacc + mrow
    inv_l = jnp.full((L,), 1.0 / L, jnp.float32)
    out_v[...] = aacc
    pltpu.sync_copy(out_v, a_hbm.at[wid])
    out_v[...] = bacc
    pltpu.sync_copy(out_v, b_hbm.at[wid])
    out_v[...] = tacc * inv_l
    pltpu.sync_copy(out_v, tok_hbm.at[wid])


@functools.cache
def _make_sc_call():
    return functools.partial(
        pl.kernel,
        mesh=plsc.VectorSubcoreMesh(core_axis_name="c", subcore_axis_name="s"),
        out_type=[
            jax.ShapeDtypeStruct((NW, L), jnp.float32),
            jax.ShapeDtypeStruct((NW, L), jnp.float32),
            jax.ShapeDtypeStruct((NW, L), jnp.float32),
        ],
        scratch_types=[
            pltpu.VMEM((BPWD * L,), jnp.int32),
            pltpu.VMEM((BPWD * L,), jnp.float32),
            pltpu.VMEM((L,), jnp.float32),
            pltpu.VMEM((SIZE,), jnp.float32),
            pltpu.VMEM((SIZE,), jnp.float32),
            pltpu.SemaphoreType.DMA,
            pltpu.SemaphoreType.DMA,
        ],
    )(_sc_kernel)


def kernel(x, target):
    target = target.astype(jnp.int32)
    tsc = target[R_TC:]
    texp = jnp.broadcast_to(tsc[:, None], (R_SC, L)).reshape(-1)
    mexp = jnp.broadcast_to(
        (tsc != PADDING_IDX).astype(jnp.float32)[:, None],
        (R_SC, L)).reshape(-1)
    a_parts, b_parts, tok_parts = _make_sc_call()(x, texp, mexp)
    a_tc, b_tc, tok_tc = _tc_part(x, target)
    a_sum = a_tc + jnp.sum(a_parts)
    b_sum = b_tc + jnp.sum(b_parts)
    tokens = tok_tc + jnp.sum(tok_parts)
    c32 = jnp.float32(C_CONST)
    return c32 - (jnp.float32(EPS) * a_sum
                  + jnp.float32(CONFIDENCE - EPS) * b_sum) / tokens


# speedup vs baseline: 2.3989x; 1.0581x over previous
import functools
import math

import jax
import jax.numpy as jnp
from jax import lax
from jax.experimental import pallas as pl
from jax.experimental.pallas import tpu as pltpu
from jax.experimental.pallas import tpu_sc as plsc

ROWS = 2048
SIZE = 32000
PADDING_IDX = 0
SMOOTHING = 0.1
CONFIDENCE = 1.0 - SMOOTHING
EPS = SMOOTHING / (SIZE - 1)
C_CONST = (SIZE - 1) * EPS * math.log(EPS) + CONFIDENCE * math.log(CONFIDENCE)

L = 16
NW = 32

R_SC = 896
R_TC = ROWS - R_SC
BPWD = R_SC // NW


RB = 128
CB = SIZE
NI = R_TC // RB


def _tc_body(t_ref, x_ref, oa_ref, ob_ref, ot_ref, acc_ref):
    i = pl.program_id(0)

    @pl.when(i == 0)
    def _init():
        acc_ref[0] = 0.0
        acc_ref[1] = 0.0
        acc_ref[2] = 0.0

    t = t_ref[0]
    mcol = t != PADDING_IDX
    mf = mcol.astype(jnp.float32)
    xb = x_ref[...]
    colid = lax.broadcasted_iota(jnp.int32, (RB, CB), 1)
    acc_ref[0] += jnp.sum(xb * mf)
    acc_ref[1] += jnp.sum(jnp.where((colid == t) & mcol, xb, 0.0))
    acc_ref[2] += jnp.sum(mf)

    @pl.when(i == NI - 1)
    def _fin():
        oa_ref[0, 0] = acc_ref[0]
        ob_ref[0, 0] = acc_ref[1]
        ot_ref[0, 0] = acc_ref[2]


def _tc_part(x, target):
    t3 = target[:R_TC].reshape(NI, RB, 1)
    scalar_spec = pl.BlockSpec((1, 1), lambda i: (0, 0),
                               memory_space=pltpu.SMEM)
    outs = pl.pallas_call(
        _tc_body,
        grid=(NI,),
        in_specs=[
            pl.BlockSpec((1, RB, 1), lambda i: (i, 0, 0)),
            pl.BlockSpec((RB, CB), lambda i: (i, 0)),
        ],
        out_specs=[scalar_spec, scalar_spec, scalar_spec],
        out_shape=[jax.ShapeDtypeStruct((1, 1), jnp.float32)] * 3,
        scratch_shapes=[pltpu.SMEM((3,), jnp.float32)],
    )(t3, x)
    return outs[0][0, 0], outs[1][0, 0], outs[2][0, 0]


def _sc_kernel(x_hbm, texp_hbm, mexp_hbm, a_hbm, b_hbm, tok_hbm,
               texp_v, mexp_v, out_v, row0, row1, sem0, sem1):
    wid = lax.axis_index("s") * 2 + lax.axis_index("c")
    vzero = jnp.zeros((L,), jnp.float32)
    lanes = lax.iota(jnp.int32, L)
    dbase = R_TC + wid * BPWD
    pltpu.sync_copy(texp_hbm.at[pl.ds(wid * BPWD * L, BPWD * L)], texp_v)
    pltpu.sync_copy(mexp_hbm.at[pl.ds(wid * BPWD * L, BPWD * L)], mexp_v)

    def row_scan(buf, t16):
        def it(k, accs):
            a0, a1, a2, a3, g = accs
            o = k * 64
            c0 = buf[pl.ds(o, L)]
            c1 = buf[pl.ds(o + 16, L)]
            c2 = buf[pl.ds(o + 32, L)]
            c3 = buf[pl.ds(o + 48, L)]
            col0 = lanes + o
            g = g + jnp.where(col0 == t16, c0, 0.0)
            g = g + jnp.where(col0 + 16 == t16, c1, 0.0)
            g = g + jnp.where(col0 + 32 == t16, c2, 0.0)
            g = g + jnp.where(col0 + 48 == t16, c3, 0.0)
            return (a0 + c0, a1 + c1, a2 + c2, a3 + c3, g)
        a0, a1, a2, a3, g = lax.fori_loop(0, SIZE // 64, it, (vzero,) * 5)
        return (a0 + a1) + (a2 + a3), g

    bufs = (row0, row1)
    sems = (sem0, sem1)
    cps = [None, None]
    cps[0] = pltpu.async_copy(x_hbm.at[dbase], row0, sem0)
    aacc = vzero
    bacc = vzero
    tacc = vzero
    for r in range(BPWD):
        b = r % 2
        if r + 1 < BPWD:
            nb = (r + 1) % 2
            cps[nb] = pltpu.async_copy(
                x_hbm.at[dbase + r + 1], bufs[nb], sems[nb])
        cps[b].wait()
        t16 = texp_v[pl.ds(r * L, L)]
        mrow = mexp_v[pl.ds(r * L, L)]
        rowvec, g16 = row_scan(bufs[b], t16)
        aacc = aacc + rowvec * mrow
        bacc = bacc + g16 * mrow
        tacc = tacc + mrow
    inv_l = jnp.full((L,), 1.0 / L, jnp.float32)
    out_v[...] = aacc
    pltpu.sync_copy(out_v, a_hbm.at[wid])
    out_v[...] = bacc
    pltpu.sync_copy(out_v, b_hbm.at[wid])
    out_v[...] = tacc * inv_l
    pltpu.sync_copy(out_v, tok_hbm.at[wid])


@functools.cache
def _make_sc_call():
    return functools.partial(
        pl.kernel,
        mesh=plsc.VectorSubcoreMesh(core_axis_name="c", subcore_axis_name="s"),
        out_type=[
            jax.ShapeDtypeStruct((NW, L), jnp.float32),
            jax.ShapeDtypeStruct((NW, L), jnp.float32),
            jax.ShapeDtypeStruct((NW, L), jnp.float32),
        ],
        scratch_types=[
            pltpu.VMEM((BPWD * L,), jnp.int32),
            pltpu.VMEM((BPWD * L,), jnp.float32),
            pltpu.VMEM((L,), jnp.float32),
            pltpu.VMEM((SIZE,), jnp.float32),
            pltpu.VMEM((SIZE,), jnp.float32),
            pltpu.SemaphoreType.DMA,
            pltpu.SemaphoreType.DMA,
        ],
    )(_sc_kernel)


def kernel(x, target):
    target = target.astype(jnp.int32)
    tsc = target[R_TC:]
    texp = jnp.broadcast_to(tsc[:, None], (R_SC, L)).reshape(-1)
    mexp = jnp.broadcast_to(
        (tsc != PADDING_IDX).astype(jnp.float32)[:, None],
        (R_SC, L)).reshape(-1)
    a_parts, b_parts, tok_parts = _make_sc_call()(x, texp, mexp)
    a_tc, b_tc, tok_tc = _tc_part(x, target)
    a_sum = a_tc + jnp.sum(a_parts)
    b_sum = b_tc + jnp.sum(b_parts)
    tokens = tok_tc + jnp.sum(tok_parts)
    c32 = jnp.float32(C_CONST)
    return c32 - (jnp.float32(EPS) * a_sum
                  + jnp.float32(CONFIDENCE - EPS) * b_sum) / tokens


# speedup vs baseline: 2.4286x; 1.0124x over previous
import functools
import math

import jax
import jax.numpy as jnp
from jax import lax
from jax.experimental import pallas as pl
from jax.experimental.pallas import tpu as pltpu
from jax.experimental.pallas import tpu_sc as plsc

ROWS = 2048
SIZE = 32000
PADDING_IDX = 0
SMOOTHING = 0.1
CONFIDENCE = 1.0 - SMOOTHING
EPS = SMOOTHING / (SIZE - 1)
C_CONST = (SIZE - 1) * EPS * math.log(EPS) + CONFIDENCE * math.log(CONFIDENCE)

L = 16
NW = 32

R_SC = 896
R_TC = ROWS - R_SC
BPWD = R_SC // NW


RB = 128
CB = SIZE
NI = R_TC // RB


def _tc_body(t_ref, x_ref, oa_ref, ob_ref, ot_ref, acc_ref):
    i = pl.program_id(0)

    @pl.when(i == 0)
    def _init():
        acc_ref[0] = 0.0
        acc_ref[1] = 0.0
        acc_ref[2] = 0.0

    t = t_ref[0]
    mcol = t != PADDING_IDX
    mf = mcol.astype(jnp.float32)
    xb = x_ref[...]
    colid = lax.broadcasted_iota(jnp.int32, (RB, CB), 1)
    acc_ref[0] += jnp.sum(xb * mf)
    acc_ref[1] += jnp.sum(jnp.where((colid == t) & mcol, xb, 0.0))
    acc_ref[2] += jnp.sum(mf)

    @pl.when(i == NI - 1)
    def _fin():
        oa_ref[0, 0] = acc_ref[0]
        ob_ref[0, 0] = acc_ref[1]
        ot_ref[0, 0] = acc_ref[2]


def _tc_part(x, target):
    t3 = target[:R_TC].reshape(NI, RB, 1)
    scalar_spec = pl.BlockSpec((1, 1), lambda i: (0, 0),
                               memory_space=pltpu.SMEM)
    outs = pl.pallas_call(
        _tc_body,
        grid=(NI,),
        in_specs=[
            pl.BlockSpec((1, RB, 1), lambda i: (i, 0, 0)),
            pl.BlockSpec((RB, CB), lambda i: (i, 0)),
        ],
        out_specs=[scalar_spec, scalar_spec, scalar_spec],
        out_shape=[jax.ShapeDtypeStruct((1, 1), jnp.float32)] * 3,
        scratch_shapes=[pltpu.SMEM((3,), jnp.float32)],
    )(t3, x)
    return outs[0][0, 0], outs[1][0, 0], outs[2][0, 0]


def _sc_kernel(x_hbm, texp_hbm, mexp_hbm, a_hbm, b_hbm, tok_hbm,
               texp_v, mexp_v, out_v, row0, row1, sem0, sem1):
    wid = lax.axis_index("s") * 2 + lax.axis_index("c")
    vzero = jnp.zeros((L,), jnp.float32)
    lanes = lax.iota(jnp.int32, L)
    dbase = R_TC + wid * BPWD
    pltpu.sync_copy(texp_hbm.at[pl.ds(wid * BPWD * L, BPWD * L)], texp_v)
    pltpu.sync_copy(mexp_hbm.at[pl.ds(wid * BPWD * L, BPWD * L)], mexp_v)

    UNROLL = 16

    def row_scan(buf, t16):
        tl = t16 - lanes

        def it(k, accs):
            a0, a1, a2, a3, g = accs
            o = k * (UNROLL * L)
            tlk = tl - o
            aa = [a0, a1, a2, a3]
            for j in range(UNROLL):
                c = buf[pl.ds(o + j * L, L)]
                aa[j % 4] = aa[j % 4] + c
                g = g + jnp.where(tlk == j * L, c, 0.0)
            return (aa[0], aa[1], aa[2], aa[3], g)
        a0, a1, a2, a3, g = lax.fori_loop(0, SIZE // (UNROLL * L), it,
                                          (vzero,) * 5)
        return (a0 + a1) + (a2 + a3), g

    bufs = (row0, row1)
    sems = (sem0, sem1)
    cps = [None, None]
    cps[0] = pltpu.async_copy(x_hbm.at[dbase], row0, sem0)
    aacc = vzero
    bacc = vzero
    tacc = vzero
    for r in range(BPWD):
        b = r % 2
        if r + 1 < BPWD:
            nb = (r + 1) % 2
            cps[nb] = pltpu.async_copy(
                x_hbm.at[dbase + r + 1], bufs[nb], sems[nb])
        cps[b].wait()
        t16 = texp_v[pl.ds(r * L, L)]
        mrow = mexp_v[pl.ds(r * L, L)]
        rowvec, g16 = row_scan(bufs[b], t16)
        aacc = aacc + rowvec * mrow
        bacc = bacc + g16 * mrow
        tacc = tacc + mrow
    inv_l = jnp.full((L,), 1.0 / L, jnp.float32)
    out_v[...] = aacc
    pltpu.sync_copy(out_v, a_hbm.at[wid])
    out_v[...] = bacc
    pltpu.sync_copy(out_v, b_hbm.at[wid])
    out_v[...] = tacc * inv_l
    pltpu.sync_copy(out_v, tok_hbm.at[wid])


@functools.cache
def _make_sc_call():
    return functools.partial(
        pl.kernel,
        mesh=plsc.VectorSubcoreMesh(core_axis_name="c", subcore_axis_name="s"),
        out_type=[
            jax.ShapeDtypeStruct((NW, L), jnp.float32),
            jax.ShapeDtypeStruct((NW, L), jnp.float32),
            jax.ShapeDtypeStruct((NW, L), jnp.float32),
        ],
        scratch_types=[
            pltpu.VMEM((BPWD * L,), jnp.int32),
            pltpu.VMEM((BPWD * L,), jnp.float32),
            pltpu.VMEM((L,), jnp.float32),
            pltpu.VMEM((SIZE,), jnp.float32),
            pltpu.VMEM((SIZE,), jnp.float32),
            pltpu.SemaphoreType.DMA,
            pltpu.SemaphoreType.DMA,
        ],
    )(_sc_kernel)


def kernel(x, target):
    target = target.astype(jnp.int32)
    tsc = target[R_TC:]
    texp = jnp.broadcast_to(tsc[:, None], (R_SC, L)).reshape(-1)
    mexp = jnp.broadcast_to(
        (tsc != PADDING_IDX).astype(jnp.float32)[:, None],
        (R_SC, L)).reshape(-1)
    a_parts, b_parts, tok_parts = _make_sc_call()(x, texp, mexp)
    a_tc, b_tc, tok_tc = _tc_part(x, target)
    a_sum = a_tc + jnp.sum(a_parts)
    b_sum = b_tc + jnp.sum(b_parts)
    tokens = tok_tc + jnp.sum(tok_parts)
    c32 = jnp.float32(C_CONST)
    return c32 - (jnp.float32(EPS) * a_sum
                  + jnp.float32(CONFIDENCE - EPS) * b_sum) / tokens
